# Initial kernel scaffold; baseline (speedup 1.0000x reference)
import jax
import jax.numpy as jnp
from jax.experimental import pallas as pl


def kernel(z, W):
    raise NotImplementedError("write your pallas kernel here")


# speedup vs baseline: 1.0348x; 1.0348x over previous
import functools

import jax
import jax.numpy as jnp
from jax import lax
from jax.experimental import pallas as pl
from jax.experimental.pallas import tpu as pltpu
from jax.experimental.pallas import tpu_sc as plsc

B_TOTAL = 8192
N_CODES = 8192
D_DIM = 32
B_BLK = 256
N_BLKS = B_TOTAL // B_BLK


def _argmin_body(w_ref, zt_ref, wsq_ref, zsq_ref, idx_ref, loss_ref):
    e = lax.dot_general(
        w_ref[...], zt_ref[...], (((1,), (0,)), ((), ())),
        preferred_element_type=jnp.float32)
    d = (zsq_ref[...] + wsq_ref[...]) - 2.0 * e
    m = jnp.min(d, axis=0, keepdims=True)
    rows = lax.broadcasted_iota(jnp.int32, d.shape, 0)
    big = jnp.int32(N_CODES)
    idx = jnp.min(jnp.where(d == m, rows, big), axis=0)
    idx_ref[0, 0, :] = idx

    @pl.when(pl.program_id(0) == 0)
    def _init():
        loss_ref[0, 0] = 0.0

    loss_ref[0, 0] += jnp.sum(m) * (1.0 / (B_TOTAL * D_DIM))


_argmin_call = pl.pallas_call(
    _argmin_body,
    grid=(N_BLKS,),
    in_specs=[
        pl.BlockSpec((N_CODES, D_DIM), lambda i: (0, 0)),
        pl.BlockSpec((D_DIM, B_BLK), lambda i: (0, i)),
        pl.BlockSpec((N_CODES, 1), lambda i: (0, 0)),
        pl.BlockSpec((1, B_BLK), lambda i: (0, i)),
    ],
    out_specs=[
        pl.BlockSpec((1, 1, B_BLK), lambda i: (i, 0, 0)),
        pl.BlockSpec(memory_space=pltpu.SMEM),
    ],
    out_shape=[
        jax.ShapeDtypeStruct((N_BLKS, 1, B_BLK), jnp.int32),
        jax.ShapeDtypeStruct((1, 1), jnp.float32),
    ],
)


@functools.lru_cache(maxsize=None)
def _build_sc_gather():
    info = plsc.get_sparse_core_info()
    nw = info.num_cores * info.num_subcores
    b_per_w = B_TOTAL // nw
    mesh = plsc.VectorSubcoreMesh(core_axis_name="c", subcore_axis_name="s")

    @functools.partial(
        pl.kernel,
        mesh=mesh,
        out_type=jax.ShapeDtypeStruct((B_TOTAL, D_DIM), jnp.float32),
        scratch_types=[
            pltpu.VMEM((b_per_w,), jnp.int32),
            pltpu.VMEM((b_per_w, D_DIM), jnp.float32),
            pltpu.SemaphoreType.DMA,
        ],
        compiler_params=pltpu.CompilerParams(use_tc_tiling_on_sc=False),
    )
    def _sc_gather(table_hbm, idx_hbm, out_hbm, idx_v, rows_v, sem):
        wid = lax.axis_index("s") * info.num_cores + lax.axis_index("c")
        base = wid * b_per_w
        pltpu.sync_copy(idx_hbm.at[pl.ds(base, b_per_w)], idx_v)
        pltpu.async_copy(table_hbm.at[idx_v], rows_v, sem).wait()
        pltpu.sync_copy(rows_v, out_hbm.at[pl.ds(base, b_per_w)])

    return _sc_gather


def kernel(z, W):
    z_flat = z.reshape(-1, D_DIM)
    zsq = jnp.sum(z_flat ** 2, axis=1, keepdims=True)
    wsq = jnp.sum(W ** 2, axis=1)[None, :]
    zbt = z_flat.astype(jnp.bfloat16).astype(jnp.float32).T
    idx_blocks, loss11 = _argmin_call(W, zbt, wsq.T, zsq.T)
    idx = idx_blocks.reshape(-1)
    z_q = _build_sc_gather()(W, idx)
    return (z_q.reshape(z.shape), loss11[0, 0], idx.reshape(z.shape[:-1]))


# speedup vs baseline: 1.0598x; 1.0241x over previous
import functools

import jax
import jax.numpy as jnp
from jax import lax
from jax.experimental import pallas as pl
from jax.experimental.pallas import tpu as pltpu
from jax.experimental.pallas import tpu_sc as plsc

B_TOTAL = 8192
N_CODES = 8192
D_DIM = 32
B_BLK = 256
N_BLKS = B_TOTAL // B_BLK


def _argmin_body(w_ref, zt_ref, wsq_ref, zsq_ref, idx_ref, loss_ref):
    e = lax.dot_general(
        w_ref[...], zt_ref[...], (((1,), (0,)), ((), ())),
        preferred_element_type=jnp.float32)
    s = wsq_ref[...] - 2.0 * e
    m = jnp.min(s, axis=0, keepdims=True)
    rows = lax.broadcasted_iota(jnp.int32, s.shape, 0)
    big = jnp.int32(N_CODES)
    idx = jnp.min(jnp.where(s == m, rows, big), axis=0)
    idx_ref[0, 0, :] = idx

    @pl.when(pl.program_id(0) == 0)
    def _init():
        loss_ref[0, 0] = 0.0

    loss_ref[0, 0] += (jnp.sum(zsq_ref[...]) + jnp.sum(m)) * (
        1.0 / (B_TOTAL * D_DIM))


_argmin_call = pl.pallas_call(
    _argmin_body,
    grid=(N_BLKS,),
    in_specs=[
        pl.BlockSpec((N_CODES, D_DIM), lambda i: (0, 0)),
        pl.BlockSpec((D_DIM, B_BLK), lambda i: (0, i)),
        pl.BlockSpec((N_CODES, 1), lambda i: (0, 0)),
        pl.BlockSpec((1, B_BLK), lambda i: (0, i)),
    ],
    out_specs=[
        pl.BlockSpec((1, 1, B_BLK), lambda i: (i, 0, 0)),
        pl.BlockSpec(memory_space=pltpu.SMEM),
    ],
    out_shape=[
        jax.ShapeDtypeStruct((N_BLKS, 1, B_BLK), jnp.int32),
        jax.ShapeDtypeStruct((1, 1), jnp.float32),
    ],
)


@functools.lru_cache(maxsize=None)
def _build_sc_gather():
    info = plsc.get_sparse_core_info()
    nw = info.num_cores * info.num_subcores
    b_per_w = B_TOTAL // nw
    mesh = plsc.VectorSubcoreMesh(core_axis_name="c", subcore_axis_name="s")

    @functools.partial(
        pl.kernel,
        mesh=mesh,
        out_type=jax.ShapeDtypeStruct((B_TOTAL, D_DIM), jnp.float32),
        scratch_types=[
            pltpu.VMEM((b_per_w,), jnp.int32),
            pltpu.VMEM((b_per_w, D_DIM), jnp.float32),
            pltpu.SemaphoreType.DMA,
        ],
        compiler_params=pltpu.CompilerParams(use_tc_tiling_on_sc=False),
    )
    def _sc_gather(table_hbm, idx_hbm, out_hbm, idx_v, rows_v, sem):
        wid = lax.axis_index("s") * info.num_cores + lax.axis_index("c")
        base = wid * b_per_w
        pltpu.sync_copy(idx_hbm.at[pl.ds(base, b_per_w)], idx_v)
        pltpu.async_copy(table_hbm.at[idx_v], rows_v, sem).wait()
        pltpu.sync_copy(rows_v, out_hbm.at[pl.ds(base, b_per_w)])

    return _sc_gather


def kernel(z, W):
    z_flat = z.reshape(-1, D_DIM)
    zsq = jnp.sum(z_flat ** 2, axis=1, keepdims=True)
    wsq = jnp.sum(W ** 2, axis=1)[None, :]
    zbt = z_flat.astype(jnp.bfloat16).astype(jnp.float32).T
    idx_blocks, loss11 = _argmin_call(W, zbt, wsq.T, zsq.T)
    idx = idx_blocks.reshape(-1)
    z_q = _build_sc_gather()(W, idx)
    return (z_q.reshape(z.shape), loss11[0, 0], idx.reshape(z.shape[:-1]))
